# baseline (speedup 1.0000x reference)
import functools
import math

import jax
import jax.numpy as jnp
import numpy as np
from jax import lax
from jax.experimental import pallas as pl
from jax.experimental.pallas import tpu as pltpu
from jax.experimental.pallas import tpu_sc as plsc

_MAX_NORM = 2.0
_GATHER_WINDOW = 128

_S1 = 9.99999981e-01
_S3 = -1.66666497e-01
_S5 = 8.33292673e-03
_S7 = -1.98022542e-04
_S9 = 2.59281518e-06
_HALF_PI = 1.5707963267948966


def _sc_gather(table, idx):
    n = idx.shape[0]
    d = table.shape[1]
    w = _GATHER_WINDOW
    assert n % w == 0
    idx2 = idx.reshape(1, n)
    mesh = plsc.VectorSubcoreMesh(core_axis_name="core", subcore_axis_name="subcore")

    @functools.partial(
        pl.kernel,
        out_type=jax.ShapeDtypeStruct((n, d), table.dtype),
        mesh=mesh,
        compiler_params=pltpu.CompilerParams(use_tc_tiling_on_sc=False),
    )
    def gather_kernel(x_hbm, i_hbm, o_hbm):
        def body(i_vmem, o_vmem):
            pltpu.sync_copy(x_hbm.at[i_vmem.at[0]], o_vmem)

        pltpu.emit_pipeline(
            body,
            grid=(n // w,),
            in_specs=[pl.BlockSpec((1, w), index_map=lambda i: (0, i))],
            out_specs=[pl.BlockSpec((w, d), index_map=lambda i: (i, 0))],
            core_axis_name=("core", "subcore"),
            dimension_semantics=(pltpu.PARALLEL,),
        )(i_hbm, o_hbm)

    return gather_kernel(table, idx2)


def _sin_poly(x):
    x2 = x * x
    return ((((_S9 * x2 + _S7) * x2 + _S5) * x2 + _S3) * x2 + _S1) * x


def _pe_half(x, t, coef):
    d = x.shape[1]
    s = jnp.sum(x * x, axis=1, keepdims=True)
    scale = jnp.where(s > _MAX_NORM * _MAX_NORM, _MAX_NORM * lax.rsqrt(s), 1.0)
    mz = x * (scale * math.sqrt(d))
    phase = t * coef
    lane = lax.broadcasted_iota(jnp.int32, (1, d), 1)
    arg = jnp.where(lane % 2 == 1, _HALF_PI - phase, phase)
    return mz + _sin_poly(arg)


def _finish_body(g_ref, te_ref, to_ref, coef_ref, out_ref):
    g = g_ref[0]
    d = g.shape[1] // 2
    coef = coef_ref[...]
    te = te_ref[0]
    to = to_ref[0]
    e = _pe_half(g[:, :d], te, coef)
    o = _pe_half(g[:, d:], to, coef)
    et = e.T
    ot = o.T
    out_ref[0] = jnp.stack([et, ot], axis=-1).reshape(d, -1)


def _tc_finish(g3, te3, to3, coef2d):
    l, half_b, dd = g3.shape
    d = dd // 2
    b = half_b * 2
    return pl.pallas_call(
        _finish_body,
        grid=(l,),
        in_specs=[
            pl.BlockSpec((1, half_b, dd), lambda i: (i, 0, 0)),
            pl.BlockSpec((1, half_b, 1), lambda i: (i, 0, 0)),
            pl.BlockSpec((1, half_b, 1), lambda i: (i, 0, 0)),
            pl.BlockSpec((1, d), lambda i: (0, 0)),
        ],
        out_specs=pl.BlockSpec((1, d, b), lambda i: (i, 0, 0)),
        out_shape=jax.ShapeDtypeStruct((l, d, b), jnp.float32),
    )(g3, te3, to3, coef2d)


def kernel(mz_batch, int_batch, table):
    b, l = mz_batch.shape
    d = table.shape[1]
    n = b * l
    idx = mz_batch.T.astype(jnp.int32).reshape(-1)
    j = np.arange(d)
    coef2d = jnp.asarray(
        (j / (10000.0 ** (2.0 * j / d))).astype(np.float32)
    ).reshape(1, d)
    gathered = _sc_gather(table, idx)
    g3 = gathered.reshape(l, b // 2, 2 * d)
    int_t = int_batch.T
    te3 = int_t[:, 0::2].reshape(l, b // 2, 1)
    to3 = int_t[:, 1::2].reshape(l, b // 2, 1)
    out3 = _tc_finish(g3, te3, to3, coef2d)
    return jnp.transpose(out3, (2, 0, 1))


# speedup vs baseline: 1.0014x; 1.0014x over previous
import functools
import math

import jax
import jax.numpy as jnp
import numpy as np
from jax import lax
from jax.experimental import pallas as pl
from jax.experimental.pallas import tpu as pltpu
from jax.experimental.pallas import tpu_sc as plsc

_MAX_NORM = 2.0
_W = 128
_GW = 5
_GR = _W * _GW

_S1 = 9.99999981e-01
_S3 = -1.66666497e-01
_S5 = 8.33292673e-03
_S7 = -1.98022542e-04
_S9 = 2.59281518e-06
_HALF_PI = 1.5707963267948966


def _sc_gather(table, idx):
    n = idx.shape[0]
    d = table.shape[1]
    info = plsc.get_sparse_core_info()
    nc, ns = info.num_cores, info.num_subcores
    nw = nc * ns
    rpw = n // nw
    assert n % nw == 0 and rpw % _GR == 0
    ngroups = rpw // _GR
    assert ngroups % 2 == 0
    mesh = plsc.VectorSubcoreMesh(core_axis_name="c", subcore_axis_name="s")

    @functools.partial(
        pl.kernel,
        out_type=jax.ShapeDtypeStruct((n, d), table.dtype),
        mesh=mesh,
        compiler_params=pltpu.CompilerParams(use_tc_tiling_on_sc=False),
        scratch_types=[
            pltpu.VMEM((rpw,), jnp.int32),
            pltpu.VMEM((_GR, d), jnp.float32),
            pltpu.VMEM((_GR, d), jnp.float32),
            pltpu.SemaphoreType.DMA,
            pltpu.SemaphoreType.DMA,
            pltpu.SemaphoreType.DMA,
        ],
    )
    def gather_kernel(x_hbm, i_hbm, o_hbm, idx_v, buf0, buf1, gsem, wsem0, wsem1):
        wid = lax.axis_index("s") * nc + lax.axis_index("c")
        base = wid * rpw
        pltpu.sync_copy(i_hbm.at[pl.ds(base, rpw)], idx_v)
        bufs = (buf0, buf1)
        wsems = (wsem0, wsem1)

        def run_group(g, p):
            buf = bufs[p]
            handles = [
                pltpu.async_copy(
                    x_hbm.at[idx_v.at[pl.ds(g * _GR + w * _W, _W)]],
                    buf.at[pl.ds(w * _W, _W)],
                    gsem,
                )
                for w in range(_GW)
            ]
            for h in handles:
                h.wait()
            return pltpu.async_copy(
                buf, o_hbm.at[pl.ds(base + g * _GR, _GR)], wsems[p]
            )

        def outer(i, carry):
            g0 = i * 2
            writes = [run_group(g0 + p, p) for p in range(2)]
            for h in writes:
                h.wait()
            return carry

        lax.fori_loop(0, ngroups // 2, outer, 0)

    return gather_kernel(table, idx)


def _sin_poly(x):
    x2 = x * x
    return ((((_S9 * x2 + _S7) * x2 + _S5) * x2 + _S3) * x2 + _S1) * x


def _pe_half(x, t, coef):
    d = x.shape[1]
    s = jnp.sum(x * x, axis=1, keepdims=True)
    scale = jnp.where(s > _MAX_NORM * _MAX_NORM, _MAX_NORM * lax.rsqrt(s), 1.0)
    mz = x * (scale * math.sqrt(d))
    phase = t * coef
    lane = lax.broadcasted_iota(jnp.int32, (1, d), 1)
    arg = jnp.where(lane % 2 == 1, _HALF_PI - phase, phase)
    return mz + _sin_poly(arg)


def _finish_body(g_ref, te_ref, to_ref, coef_ref, out_ref):
    g = g_ref[0]
    d = g.shape[1] // 2
    coef = coef_ref[...]
    te = te_ref[0]
    to = to_ref[0]
    e = _pe_half(g[:, :d], te, coef)
    o = _pe_half(g[:, d:], to, coef)
    et = e.T
    ot = o.T
    out_ref[0] = jnp.stack([et, ot], axis=-1).reshape(d, -1)


def _tc_finish(g3, te3, to3, coef2d):
    l, half_b, dd = g3.shape
    d = dd // 2
    b = half_b * 2
    return pl.pallas_call(
        _finish_body,
        grid=(l,),
        in_specs=[
            pl.BlockSpec((1, half_b, dd), lambda i: (i, 0, 0)),
            pl.BlockSpec((1, half_b, 1), lambda i: (i, 0, 0)),
            pl.BlockSpec((1, half_b, 1), lambda i: (i, 0, 0)),
            pl.BlockSpec((1, d), lambda i: (0, 0)),
        ],
        out_specs=pl.BlockSpec((1, d, b), lambda i: (i, 0, 0)),
        out_shape=jax.ShapeDtypeStruct((l, d, b), jnp.float32),
    )(g3, te3, to3, coef2d)


def kernel(mz_batch, int_batch, table):
    b, l = mz_batch.shape
    d = table.shape[1]
    n = b * l
    idx = mz_batch.T.astype(jnp.int32).reshape(-1)
    j = np.arange(d)
    coef2d = jnp.asarray(
        (j / (10000.0 ** (2.0 * j / d))).astype(np.float32)
    ).reshape(1, d)
    gathered = _sc_gather(table, idx)
    g3 = gathered.reshape(l, b // 2, 2 * d)
    int_t = int_batch.T
    te3 = int_t[:, 0::2].reshape(l, b // 2, 1)
    to3 = int_t[:, 1::2].reshape(l, b // 2, 1)
    out3 = _tc_finish(g3, te3, to3, coef2d)
    return jnp.transpose(out3, (2, 0, 1))


# speedup vs baseline: 3.7912x; 3.7859x over previous
import functools
import math

import jax
import jax.numpy as jnp
import numpy as np
from jax import lax
from jax.experimental import pallas as pl
from jax.experimental.pallas import tpu as pltpu
from jax.experimental.pallas import tpu_sc as plsc

_MAX_NORM = 2.0
_W = 64
_GW = 5
_GR = _W * _GW

_S1 = 9.99999981e-01
_S3 = -1.66666497e-01
_S5 = 8.33292673e-03
_S7 = -1.98022542e-04
_S9 = 2.59281518e-06
_HALF_PI = 1.5707963267948966

_ROWS_PER_STEP = 1024


def _sc_gather_pairs(table, idx_even, idx_odd):
    m = idx_even.shape[0]
    d = table.shape[1]
    info = plsc.get_sparse_core_info()
    nc, ns = info.num_cores, info.num_subcores
    nw = nc * ns
    rpw = m // nw
    assert m % nw == 0 and rpw % _GR == 0
    ngroups = rpw // _GR
    assert ngroups % 2 == 0
    mesh = plsc.VectorSubcoreMesh(core_axis_name="c", subcore_axis_name="s")

    @functools.partial(
        pl.kernel,
        out_type=jax.ShapeDtypeStruct((m, 2 * d), table.dtype),
        mesh=mesh,
        compiler_params=pltpu.CompilerParams(use_tc_tiling_on_sc=False),
        scratch_types=[
            pltpu.VMEM((rpw,), jnp.int32),
            pltpu.VMEM((rpw,), jnp.int32),
            pltpu.VMEM((_GR, d), jnp.float32),
            pltpu.VMEM((_GR, d), jnp.float32),
            pltpu.VMEM((_GR, d), jnp.float32),
            pltpu.VMEM((_GR, d), jnp.float32),
            pltpu.SemaphoreType.DMA,
            pltpu.SemaphoreType.DMA,
            pltpu.SemaphoreType.DMA,
        ],
    )
    def gather_kernel(
        x_hbm, ie_hbm, io_hbm, o_hbm,
        idx_e, idx_o, bufe0, bufo0, bufe1, bufo1, gsem, wsem0, wsem1,
    ):
        wid = lax.axis_index("s") * nc + lax.axis_index("c")
        base = wid * rpw
        pltpu.sync_copy(ie_hbm.at[pl.ds(base, rpw)], idx_e)
        pltpu.sync_copy(io_hbm.at[pl.ds(base, rpw)], idx_o)
        bufs = ((bufe0, bufo0), (bufe1, bufo1))
        wsems = (wsem0, wsem1)

        def run_group(g, p):
            bufe, bufo = bufs[p]
            handles = []
            for w in range(_GW):
                sl = pl.ds(g * _GR + w * _W, _W)
                rows = pl.ds(w * _W, _W)
                handles.append(pltpu.async_copy(
                    x_hbm.at[idx_e.at[sl]], bufe.at[rows], gsem,
                ))
                handles.append(pltpu.async_copy(
                    x_hbm.at[idx_o.at[sl]], bufo.at[rows], gsem,
                ))
            for h in handles:
                h.wait()
            orows = pl.ds(base + g * _GR, _GR)
            return (
                pltpu.async_copy(
                    bufe, o_hbm.at[orows, pl.ds(0, d)], wsems[p]
                ),
                pltpu.async_copy(
                    bufo, o_hbm.at[orows, pl.ds(d, d)], wsems[p]
                ),
            )

        def outer(i, carry):
            g0 = i * 2
            writes = [run_group(g0 + p, p) for p in range(2)]
            for pair in writes:
                for h in pair:
                    h.wait()
            return carry

        lax.fori_loop(0, ngroups // 2, outer, 0)

    return gather_kernel(table, idx_even, idx_odd)


def _sin_poly(x):
    x2 = x * x
    return ((((_S9 * x2 + _S7) * x2 + _S5) * x2 + _S3) * x2 + _S1) * x


def _finish_body(g_ref, te_ref, to_ref, coef_ref, mh_ref, out_ref):
    x = g_ref[...]
    r = x.shape[0]
    dd = x.shape[1]
    rl = r // 128

    def expand(v):
        v3 = v.reshape(rl, 1, 128)
        return jnp.broadcast_to(v3, (rl, 128, 128)).reshape(r, 128)

    sub = lax.broadcasted_iota(jnp.int32, (r, dd), 0) % dd
    lane_r = lax.broadcasted_iota(jnp.int32, (r, dd), 1)
    msel = (sub == lane_r).astype(jnp.float32)
    lane_sq = lax.broadcasted_iota(jnp.int32, (dd, dd), 1)
    jleft = (lane_sq < dd // 2).astype(jnp.float32)
    jright = (lane_sq >= dd // 2).astype(jnp.float32)
    t = jnp.dot(expand(te_ref[...]) * msel, jleft,
                preferred_element_type=jnp.float32) + \
        jnp.dot(expand(to_ref[...]) * msel, jright,
                preferred_element_type=jnp.float32)
    lane = lax.broadcasted_iota(jnp.int32, (1, dd), 1)
    x2 = x * x
    s = jnp.dot(x2, mh_ref[...], preferred_element_type=jnp.float32)
    scale = jnp.where(
        s > _MAX_NORM * _MAX_NORM, _MAX_NORM * lax.rsqrt(s), 1.0
    ) * math.sqrt(dd // 2)
    phase = t * coef_ref[...]
    arg = jnp.where(lane % 2 == 1, _HALF_PI - phase, phase)
    out_ref[...] = x * scale + _sin_poly(arg)


def _tc_finish(gp, te2, to2, coef2d, mhalf):
    m, dd = gp.shape
    r = _ROWS_PER_STEP
    rl = r // 128
    return pl.pallas_call(
        _finish_body,
        grid=(m // r,),
        in_specs=[
            pl.BlockSpec((r, dd), lambda i: (i, 0)),
            pl.BlockSpec((rl, 128), lambda i: (i, 0)),
            pl.BlockSpec((rl, 128), lambda i: (i, 0)),
            pl.BlockSpec((1, dd), lambda i: (0, 0)),
            pl.BlockSpec((dd, dd), lambda i: (0, 0)),
        ],
        out_specs=pl.BlockSpec((r, dd), lambda i: (i, 0)),
        out_shape=jax.ShapeDtypeStruct((m, dd), jnp.float32),
    )(gp, te2, to2, coef2d, mhalf)


def kernel(mz_batch, int_batch, table):
    b, l = mz_batch.shape
    d = table.shape[1]
    n = b * l
    idx2 = mz_batch.reshape(n // 2, 2).astype(jnp.int32)
    j = np.arange(2 * d)
    jm = j % d
    coef2d = jnp.asarray(
        (jm / (10000.0 ** (2.0 * jm / d))).astype(np.float32)
    ).reshape(1, 2 * d)
    half = (j < d)
    mhalf = jnp.asarray(
        (half[:, None] == half[None, :]).astype(np.float32)
    )
    int2 = int_batch.reshape(n // 2, 2)
    te2 = int2[:, 0].reshape(-1, 128)
    to2 = int2[:, 1].reshape(-1, 128)
    gp = _sc_gather_pairs(table, idx2[:, 0], idx2[:, 1])
    out = _tc_finish(gp, te2, to2, coef2d, mhalf)
    return out.reshape(b, l, d)


# speedup vs baseline: 4.5278x; 1.1943x over previous
import functools
import math

import jax
import jax.numpy as jnp
import numpy as np
from jax import lax
from jax.experimental import pallas as pl
from jax.experimental.pallas import tpu as pltpu
from jax.experimental.pallas import tpu_sc as plsc

_MAX_NORM = 2.0
_W = 64
_GW = 5
_GR = _W * _GW

_S1 = 9.99999981e-01
_S3 = -1.66666497e-01
_S5 = 8.33292673e-03
_S7 = -1.98022542e-04
_S9 = 2.59281518e-06
_HALF_PI = 1.5707963267948966

_ROWS_PER_STEP = 1024


def _sc_gather_pairs(table, idx_even, idx_odd):
    m = idx_even.shape[0]
    d = table.shape[1]
    info = plsc.get_sparse_core_info()
    nc, ns = info.num_cores, info.num_subcores
    nw = nc * ns
    rpw = m // nw
    assert m % nw == 0 and rpw % _GR == 0
    ngroups = rpw // _GR
    assert ngroups % 2 == 0
    mesh = plsc.VectorSubcoreMesh(core_axis_name="c", subcore_axis_name="s")

    @functools.partial(
        pl.kernel,
        out_type=jax.ShapeDtypeStruct((m, 2 * d), table.dtype),
        mesh=mesh,
        compiler_params=pltpu.CompilerParams(use_tc_tiling_on_sc=False),
        scratch_types=[
            pltpu.VMEM((rpw,), jnp.int32),
            pltpu.VMEM((rpw,), jnp.int32),
            pltpu.VMEM((_GR, d), jnp.float32),
            pltpu.VMEM((_GR, d), jnp.float32),
            pltpu.VMEM((_GR, d), jnp.float32),
            pltpu.VMEM((_GR, d), jnp.float32),
            pltpu.SemaphoreType.DMA,
            pltpu.SemaphoreType.DMA,
            pltpu.SemaphoreType.DMA,
        ],
    )
    def gather_kernel(
        x_hbm, ie_hbm, io_hbm, o_hbm,
        idx_e, idx_o, bufe0, bufo0, bufe1, bufo1, gsem, wsem0, wsem1,
    ):
        wid = lax.axis_index("s") * nc + lax.axis_index("c")
        base = wid * rpw
        pltpu.sync_copy(ie_hbm.at[pl.ds(base, rpw)], idx_e)
        pltpu.sync_copy(io_hbm.at[pl.ds(base, rpw)], idx_o)
        bufs = ((bufe0, bufo0), (bufe1, bufo1))
        wsems = (wsem0, wsem1)

        def run_group(g, p):
            bufe, bufo = bufs[p]
            handles = []
            for w in range(_GW):
                sl = pl.ds(g * _GR + w * _W, _W)
                rows = pl.ds(w * _W, _W)
                handles.append(pltpu.async_copy(
                    x_hbm.at[idx_e.at[sl]], bufe.at[rows], gsem,
                ))
                handles.append(pltpu.async_copy(
                    x_hbm.at[idx_o.at[sl]], bufo.at[rows], gsem,
                ))
            for h in handles:
                h.wait()
            orows = pl.ds(base + g * _GR, _GR)
            return (
                pltpu.async_copy(
                    bufe, o_hbm.at[orows, pl.ds(0, d)], wsems[p]
                ),
                pltpu.async_copy(
                    bufo, o_hbm.at[orows, pl.ds(d, d)], wsems[p]
                ),
            )

        def outer(i, carry):
            g0 = i * 2
            writes = [run_group(g0 + p, p) for p in range(2)]
            for pair in writes:
                for h in pair:
                    h.wait()
            return carry

        lax.fori_loop(0, ngroups // 2, outer, 0)

    return gather_kernel(table, idx_even, idx_odd)


def _sin_poly(x):
    x2 = x * x
    return ((((_S9 * x2 + _S7) * x2 + _S5) * x2 + _S3) * x2 + _S1) * x


def _finish_body(g_ref, te_ref, to_ref, coef_ref, mh_ref, out_ref):
    x = g_ref[...]
    r = x.shape[0]
    dd = x.shape[1]
    rl = r // 128

    def expand(v):
        v3 = v.reshape(rl, 1, 128)
        return jnp.broadcast_to(v3, (rl, 128, 128)).reshape(r, 128)

    sub = lax.broadcasted_iota(jnp.int32, (r, dd), 0) % dd
    lane_r = lax.broadcasted_iota(jnp.int32, (r, dd), 1)
    msel = (sub == lane_r).astype(jnp.float32)
    lane_sq = lax.broadcasted_iota(jnp.int32, (dd, dd), 1)
    jleft = (lane_sq < dd // 2).astype(jnp.float32)
    jright = (lane_sq >= dd // 2).astype(jnp.float32)
    t = jnp.dot(expand(te_ref[...]) * msel, jleft,
                preferred_element_type=jnp.float32) + \
        jnp.dot(expand(to_ref[...]) * msel, jright,
                preferred_element_type=jnp.float32)
    lane = lax.broadcasted_iota(jnp.int32, (1, dd), 1)
    x2 = x * x
    s = jnp.dot(x2, mh_ref[...], preferred_element_type=jnp.float32)
    scale = jnp.where(
        s > _MAX_NORM * _MAX_NORM, _MAX_NORM * lax.rsqrt(s), 1.0
    ) * math.sqrt(dd // 2)
    phase = t * coef_ref[...]
    arg = jnp.where(lane % 2 == 1, _HALF_PI - phase, phase)
    res = x * scale + _sin_poly(arg)
    out_ref[0, ...] = res[:, : dd // 2]
    out_ref[1, ...] = res[:, dd // 2 :]


def _tc_finish(gp, te2, to2, coef2d, mhalf):
    m, dd = gp.shape
    r = _ROWS_PER_STEP
    rl = r // 128
    return pl.pallas_call(
        _finish_body,
        grid=(m // r,),
        in_specs=[
            pl.BlockSpec((r, dd), lambda i: (i, 0)),
            pl.BlockSpec((rl, 128), lambda i: (i, 0)),
            pl.BlockSpec((rl, 128), lambda i: (i, 0)),
            pl.BlockSpec((1, dd), lambda i: (0, 0)),
            pl.BlockSpec((dd, dd), lambda i: (0, 0)),
        ],
        out_specs=pl.BlockSpec((2, r, dd // 2), lambda i: (0, i, 0)),
        out_shape=jax.ShapeDtypeStruct((2, m, dd // 2), jnp.float32),
    )(gp, te2, to2, coef2d, mhalf)


def kernel(mz_batch, int_batch, table):
    b, l = mz_batch.shape
    d = table.shape[1]
    n = b * l
    m = n // 2
    flat_idx = mz_batch.reshape(-1).astype(jnp.int32)
    idx_e = flat_idx[:m]
    idx_o = flat_idx[m:]
    j = np.arange(2 * d)
    jm = j % d
    coef2d = jnp.asarray(
        (jm / (10000.0 ** (2.0 * jm / d))).astype(np.float32)
    ).reshape(1, 2 * d)
    half = (j < d)
    mhalf = jnp.asarray(
        (half[:, None] == half[None, :]).astype(np.float32)
    )
    int_flat = int_batch.reshape(-1)
    te2 = int_flat[:m].reshape(-1, 128)
    to2 = int_flat[m:].reshape(-1, 128)
    gp = _sc_gather_pairs(table, idx_e, idx_o)
    out = _tc_finish(gp, te2, to2, coef2d, mhalf)
    return out.reshape(b, l, d)
